# Initial kernel scaffold; baseline (speedup 1.0000x reference)
import jax
import jax.numpy as jnp
from jax.experimental import pallas as pl


def kernel(pred_scores, pred_offsets, H, W, M, target):
    raise NotImplementedError("write your pallas kernel here")


# speedup vs baseline: 25.0854x; 25.0854x over previous
import functools

import jax
import jax.numpy as jnp
from jax import lax
from jax.experimental import pallas as pl
from jax.experimental.pallas import tpu as pltpu
from jax.experimental.pallas import tpu_sc as plsc

_ALPHA = 0.6
M_S = 4
K = 16
NUM_WORKERS = 32


def _dense_body(s_ref, t_ref, q4_ref, df_ref, s0_ref, np_ref):
    s = s_ref[...]
    t = t_ref[...]
    p = jax.nn.sigmoid(s)
    ce0 = jnp.maximum(s, 0.0) + jnp.log1p(jnp.exp(-jnp.abs(s)))
    loss0 = (1.0 - _ALPHA) * ce0 * p * p
    one_m_p = 1.0 - p
    loss1 = _ALPHA * (ce0 - s) * one_m_p * one_m_p
    q4_ref[...] = jnp.sqrt(jnp.sqrt(jnp.maximum(one_m_p, 0.0)))
    df_ref[...] = loss1 - loss0
    @pl.when(pl.program_id(0) == 0)
    def _():
        s0_ref[0, 0] = 0.0
        np_ref[0, 0] = 0.0

    s0_ref[0, 0] += jnp.sum(loss0)
    np_ref[0, 0] += jnp.sum(t)


def _dense_stage(s2d, t2d, grid):
    rows = s2d.shape[0] // grid
    q4, df, s0p, npp = pl.pallas_call(
        _dense_body,
        grid=(grid,),
        in_specs=[
            pl.BlockSpec((rows, 128), lambda i: (i, 0)),
            pl.BlockSpec((rows, 128), lambda i: (i, 0)),
        ],
        out_specs=[
            pl.BlockSpec((rows, 128), lambda i: (i, 0)),
            pl.BlockSpec((rows, 128), lambda i: (i, 0)),
            pl.BlockSpec((1, 1), lambda i: (0, 0), memory_space=pltpu.SMEM),
            pl.BlockSpec((1, 1), lambda i: (0, 0), memory_space=pltpu.SMEM),
        ],
        out_shape=[
            jax.ShapeDtypeStruct(s2d.shape, jnp.float32),
            jax.ShapeDtypeStruct(s2d.shape, jnp.float32),
            jax.ShapeDtypeStruct((1, 1), jnp.float32),
            jax.ShapeDtypeStruct((1, 1), jnp.float32),
        ],
    )(s2d, t2d)
    return q4, df, s0p, npp


def _make_greedy_kernel(bpw):
    rows = bpw * K // 128
    mesh = plsc.VectorSubcoreMesh(core_axis_name="c", subcore_axis_name="s")

    r = [(i - (M_S - 1) / 2.0) / float(M_S) for i in range(M_S)]
    gy = [r[j // M_S] for j in range(K)]
    gx = [r[j % M_S] for j in range(K)]

    @functools.partial(
        pl.kernel,
        mesh=mesh,
        out_type=jax.ShapeDtypeStruct((NUM_WORKERS, K), jnp.float32),
        scratch_types=[
            pltpu.VMEM((rows, 128), jnp.float32),
            pltpu.VMEM((rows, 128), jnp.float32),
            pltpu.VMEM((rows, 128), jnp.float32),
            pltpu.VMEM((rows, 128), jnp.float32),
            pltpu.VMEM((rows, 128), jnp.float32),
            pltpu.VMEM((K,), jnp.float32),
        ],
    )
    def greedy(q4_hbm, df_hbm, oy_hbm, ox_hbm, t_hbm, out_hbm,
               q4_v, df_v, oy_v, ox_v, t_v, acc_v):
        wid = lax.axis_index("s") * 2 + lax.axis_index("c")
        pltpu.sync_copy(q4_hbm.at[wid], q4_v)
        pltpu.sync_copy(df_hbm.at[wid], df_v)
        pltpu.sync_copy(oy_hbm.at[wid], oy_v)
        pltpu.sync_copy(ox_hbm.at[wid], ox_v)
        pltpu.sync_copy(t_hbm.at[wid], t_v)

        iota = lax.iota(jnp.int32, K)
        iota_f = iota.astype(jnp.float32)
        big = jnp.float32(1e30)
        xor_idx = [jnp.bitwise_xor(iota, jnp.int32(s)) for s in (1, 2, 4, 8)]

        def _perm(x, idx):
            return lax.gather(
                x, idx[:, None],
                dimension_numbers=lax.GatherDimensionNumbers(
                    offset_dims=(), collapsed_slice_dims=(0,),
                    start_index_map=(0,)),
                slice_sizes=(1,),
                mode=lax.GatherScatterMode.PROMISE_IN_BOUNDS)

        def _allmin(x):
            for idx in xor_idx:
                x = jnp.minimum(x, _perm(x, idx))
            return x

        def block_body(b, accv):
            r = b // 8
            c = (b % 8) * K
            qv = q4_v[r, pl.ds(c, K)]
            dfv = df_v[r, pl.ds(c, K)]
            oyv = oy_v[r, pl.ds(c, K)]
            oxv = ox_v[r, pl.ds(c, K)]
            tv = t_v[r, pl.ds(c, K)]
            used = jnp.zeros((K,), jnp.float32)
            for j in range(K):
                d = jnp.abs(oyv - gy[j]) + jnp.abs(oxv - gx[j])
                keyv = d * qv + used * big
                minv = _allmin(keyv)
                notmin = jnp.sign(keyv - minv)
                kstar = _allmin(iota_f + notmin * jnp.float32(K))
                eqk = 1.0 - jnp.abs(jnp.sign(iota_f - kstar))
                sel = eqk * tv[j]
                used = used + sel
                accv = accv + sel * (dfv + 10.0 * d)
            return accv

        accv = lax.fori_loop(0, bpw, block_body,
                             jnp.zeros((K,), jnp.float32))
        acc_v[...] = accv
        pltpu.sync_copy(acc_v, out_hbm.at[wid])

    return greedy


def kernel(pred_scores, pred_offsets, H, W, M, target):
    B, HW, _ = pred_scores.shape
    n_blocks = B * HW
    h_s = target.shape[1] // M_S
    w_s = target.shape[2] // M_S

    t = (target > 0).astype(jnp.float32).reshape(B, h_s, M_S, w_s, M_S)
    t = jnp.transpose(t, (0, 1, 3, 2, 4)).reshape(n_blocks, K)
    s = pred_scores.reshape(n_blocks, K)
    oy = pred_offsets[..., 0].reshape(n_blocks, K)
    ox = pred_offsets[..., 1].reshape(n_blocks, K)

    quantum = NUM_WORKERS * 8 * 128 // K
    n_pad = -(-n_blocks // quantum) * quantum
    extra = n_pad - n_blocks
    if extra:
        s = jnp.pad(s, ((0, extra), (0, 0)), constant_values=-100.0)
        t = jnp.pad(t, ((0, extra), (0, 0)))
        oy = jnp.pad(oy, ((0, extra), (0, 0)))
        ox = jnp.pad(ox, ((0, extra), (0, 0)))

    ncols = (n_pad * K) // 128
    grid = 5
    q4, df, s0p, npp = _dense_stage(
        s.reshape(ncols, 128), t.reshape(ncols, 128), grid)

    bpw = n_pad // NUM_WORKERS
    sc_shape = (NUM_WORKERS, bpw * K // 128, 128)
    partials = _make_greedy_kernel(bpw)(
        q4.reshape(sc_shape), df.reshape(sc_shape), oy.reshape(sc_shape),
        ox.reshape(sc_shape), t.reshape(sc_shape))

    npos = jnp.maximum(npp.sum(), 1.0)
    return (s0p.sum() + partials.sum()) / npos


# speedup vs baseline: 27.4998x; 1.0962x over previous
import functools

import jax
import jax.numpy as jnp
from jax import lax
from jax.experimental import pallas as pl
from jax.experimental.pallas import tpu as pltpu
from jax.experimental.pallas import tpu_sc as plsc

_ALPHA = 0.6
M_S = 4
K = 16
NUM_WORKERS = 32


def _dense_body(s_ref, t_ref, q4_ref, df_ref, s0_ref, np_ref):
    s = s_ref[...]
    t = t_ref[...]
    p = jax.nn.sigmoid(s)
    ce0 = jnp.maximum(s, 0.0) + jnp.log1p(jnp.exp(-jnp.abs(s)))
    loss0 = (1.0 - _ALPHA) * ce0 * p * p
    one_m_p = 1.0 - p
    loss1 = _ALPHA * (ce0 - s) * one_m_p * one_m_p
    q4_ref[...] = jnp.sqrt(jnp.sqrt(jnp.maximum(one_m_p, 0.0)))
    df_ref[...] = loss1 - loss0
    @pl.when(pl.program_id(0) == 0)
    def _():
        s0_ref[0, 0] = 0.0
        np_ref[0, 0] = 0.0

    s0_ref[0, 0] += jnp.sum(loss0)
    np_ref[0, 0] += jnp.sum(t)


def _dense_stage(s2d, t2d, grid):
    rows = s2d.shape[0] // grid
    q4, df, s0p, npp = pl.pallas_call(
        _dense_body,
        grid=(grid,),
        in_specs=[
            pl.BlockSpec((rows, 128), lambda i: (i, 0)),
            pl.BlockSpec((rows, 128), lambda i: (i, 0)),
        ],
        out_specs=[
            pl.BlockSpec((rows, 128), lambda i: (i, 0)),
            pl.BlockSpec((rows, 128), lambda i: (i, 0)),
            pl.BlockSpec((1, 1), lambda i: (0, 0), memory_space=pltpu.SMEM),
            pl.BlockSpec((1, 1), lambda i: (0, 0), memory_space=pltpu.SMEM),
        ],
        out_shape=[
            jax.ShapeDtypeStruct(s2d.shape, jnp.float32),
            jax.ShapeDtypeStruct(s2d.shape, jnp.float32),
            jax.ShapeDtypeStruct((1, 1), jnp.float32),
            jax.ShapeDtypeStruct((1, 1), jnp.float32),
        ],
    )(s2d, t2d)
    return q4, df, s0p, npp


def _make_greedy_kernel(bpw):
    rows = bpw * K // 128
    mesh = plsc.VectorSubcoreMesh(core_axis_name="c", subcore_axis_name="s")

    goff = [(i - (M_S - 1) / 2.0) / float(M_S) for i in range(M_S)]

    @functools.partial(
        pl.kernel,
        mesh=mesh,
        out_type=jax.ShapeDtypeStruct((NUM_WORKERS, K), jnp.float32),
        scratch_types=[
            pltpu.VMEM((rows, 128), jnp.float32),
            pltpu.VMEM((rows, 128), jnp.float32),
            pltpu.VMEM((rows, 128), jnp.float32),
            pltpu.VMEM((rows, 128), jnp.float32),
            pltpu.VMEM((rows, 128), jnp.float32),
            pltpu.VMEM((K,), jnp.float32),
        ],
    )
    def greedy(q4_hbm, df_hbm, oy_hbm, ox_hbm, t_hbm, out_hbm,
               q4_v, df_v, oy_v, ox_v, t_v, acc_v):
        wid = lax.axis_index("s") * 2 + lax.axis_index("c")
        pltpu.sync_copy(q4_hbm.at[wid], q4_v)
        pltpu.sync_copy(df_hbm.at[wid], df_v)
        pltpu.sync_copy(oy_hbm.at[wid], oy_v)
        pltpu.sync_copy(ox_hbm.at[wid], ox_v)
        pltpu.sync_copy(t_hbm.at[wid], t_v)

        iota = lax.iota(jnp.int32, K)
        iota_f = iota.astype(jnp.float32)
        big = jnp.float32(1e30)
        xor_idx = [jnp.bitwise_xor(iota, jnp.int32(s)) for s in (1, 2, 4, 8)]

        def _perm(x, idx):
            return lax.gather(
                x, idx[:, None],
                dimension_numbers=lax.GatherDimensionNumbers(
                    offset_dims=(), collapsed_slice_dims=(0,),
                    start_index_map=(0,)),
                slice_sizes=(1,),
                mode=lax.GatherScatterMode.PROMISE_IN_BOUNDS)

        def _allmin(x):
            for idx in xor_idx:
                x = jnp.minimum(x, _perm(x, idx))
            return x

        def block_body(b, accv):
            r = b // 8
            c = (b % 8) * K
            qv = q4_v[r, pl.ds(c, K)]
            dfv = df_v[r, pl.ds(c, K)]
            oyv = oy_v[r, pl.ds(c, K)]
            oxv = ox_v[r, pl.ds(c, K)]
            tv = t_v[r, pl.ds(c, K)]
            dys = [jnp.abs(oyv - g) for g in goff]
            dxs = [jnp.abs(oxv - g) for g in goff]
            dyqs = [dy * qv for dy in dys]
            dxqs = [dx * qv for dx in dxs]
            used = jnp.zeros((K,), jnp.float32)
            for j in range(K):
                jy, jx = j // M_S, j % M_S
                keyv = dyqs[jy] + dxqs[jx] + used * big
                minv = _allmin(keyv)
                sel = (1.0 - jnp.sign(keyv - minv)) * tv[j]
                used = used + sel
                accv = accv + sel * (dfv + 10.0 * (dys[jy] + dxs[jx]))
            return accv

        accv = lax.fori_loop(0, bpw, block_body,
                             jnp.zeros((K,), jnp.float32))
        acc_v[...] = accv
        pltpu.sync_copy(acc_v, out_hbm.at[wid])

    return greedy


def kernel(pred_scores, pred_offsets, H, W, M, target):
    B, HW, _ = pred_scores.shape
    n_blocks = B * HW
    h_s = target.shape[1] // M_S
    w_s = target.shape[2] // M_S

    t = (target > 0).astype(jnp.float32).reshape(B, h_s, M_S, w_s, M_S)
    t = jnp.transpose(t, (0, 1, 3, 2, 4)).reshape(n_blocks, K)
    s = pred_scores.reshape(n_blocks, K)
    oy = pred_offsets[..., 0].reshape(n_blocks, K)
    ox = pred_offsets[..., 1].reshape(n_blocks, K)

    quantum = NUM_WORKERS * 8 * 128 // K
    n_pad = -(-n_blocks // quantum) * quantum
    extra = n_pad - n_blocks
    if extra:
        s = jnp.pad(s, ((0, extra), (0, 0)), constant_values=-100.0)
        t = jnp.pad(t, ((0, extra), (0, 0)))
        oy = jnp.pad(oy, ((0, extra), (0, 0)))
        ox = jnp.pad(ox, ((0, extra), (0, 0)))

    ncols = (n_pad * K) // 128
    grid = 5
    q4, df, s0p, npp = _dense_stage(
        s.reshape(ncols, 128), t.reshape(ncols, 128), grid)

    bpw = n_pad // NUM_WORKERS
    sc_shape = (NUM_WORKERS, bpw * K // 128, 128)
    partials = _make_greedy_kernel(bpw)(
        q4.reshape(sc_shape), df.reshape(sc_shape), oy.reshape(sc_shape),
        ox.reshape(sc_shape), t.reshape(sc_shape))

    npos = jnp.maximum(npp.sum(), 1.0)
    return (s0p.sum() + partials.sum()) / npos
